# baseline (speedup 1.0000x reference)
import functools

import jax
import jax.numpy as jnp
from jax import lax
from jax.experimental import pallas as pl
from jax.experimental.pallas import tpu as pltpu
from jax.experimental.pallas import tpu_sc as plsc

N_LOCS = 640000
N_CLUSTERS = 10000
N_FOV = 16
HID = 64
N_CLASSES = 4

P = 32
NR = N_LOCS // P
RT = 400
C = 16
IDS_W = 1024
NR_PAD = 20480
NW = 32
CPT = 320
NC_PAD = NW * CPT
TROWS = CPT + 16


def _mlp_body(x_ref, p_ref, w1x_ref, w1p_ref, b1_ref, w2_ref, b2_ref, o_ref):
    h1 = jnp.dot(x_ref[...], w1x_ref[...], preferred_element_type=jnp.float32)
    h1 = h1 + jnp.dot(p_ref[...], w1p_ref[...], preferred_element_type=jnp.float32)
    h1 = jnp.maximum(h1 + b1_ref[0:1, :], 0.0).astype(jnp.bfloat16)
    for jj in range(8):
        hjj = h1[:, 256 * jj:256 * (jj + 1)]
        o = jnp.dot(hjj, w2_ref[...], preferred_element_type=jnp.float32)
        o = jnp.maximum(o + b2_ref[0:1, :], 0.0).astype(jnp.bfloat16)
        o_ref[2 * jj, :, :] = o[:, 0:128]
        o_ref[2 * jj + 1, :, :] = o[:, 128:256]


def _run_mlp(xr, pr, w1x, w1p, b1t, w2bd, b2t):
    grid = (NR // RT,)
    return pl.pallas_call(
        _mlp_body,
        grid=grid,
        in_specs=[
            pl.BlockSpec((RT, 4 * P), lambda i: (i, 0)),
            pl.BlockSpec((RT, 3 * P), lambda i: (i, 0)),
            pl.BlockSpec((4 * P, HID * P), lambda i: (0, 0)),
            pl.BlockSpec((3 * P, HID * P), lambda i: (0, 0)),
            pl.BlockSpec((8, HID * P), lambda i: (0, 0)),
            pl.BlockSpec((256, 256), lambda i: (0, 0)),
            pl.BlockSpec((8, 256), lambda i: (0, 0)),
        ],
        out_specs=pl.BlockSpec((16, RT, 128), lambda i: (0, i, 0)),
        out_shape=jax.ShapeDtypeStruct((16, NR, 128), jnp.bfloat16),
    )(xr, pr, w1x, w1p, b1t, w2bd, b2t)


def _segmax_body(h2_hbm, ids_hbm, bounds_hbm, out_hbm, bounds_v, ids_v, rows_v, table_v):
    w = lax.axis_index("s") * 2 + lax.axis_index("c")
    pltpu.sync_copy(bounds_hbm, bounds_v)
    bv = bounds_v[pl.ds(w, 16)]
    row_lo = bv[0]
    row_hi = bv[1]
    lo32 = lax.div(row_lo, jnp.int32(P))
    hi32 = lax.div(row_hi + (P - 1), jnp.int32(P))
    base = lax.bitwise_and(lo32, jnp.int32(-16))
    c_lo = w * CPT

    def _zero(i, carry):
        for f in range(HID // 32):
            table_v[i, 0, pl.ds(32 * f, 32)] = jnp.zeros((32,), jnp.bfloat16)
            table_v[i, 1, pl.ds(32 * f, 32)] = jnp.zeros((32,), jnp.bfloat16)
        return carry

    lax.fori_loop(0, TROWS, _zero, 0)

    nk = (hi32 - base + (C - 1)) // C
    zero32 = jnp.zeros((32,), jnp.bfloat16)

    def _flush(prev, accs):
        for f in range(HID // 32):
            sl = pl.ds(32 * f, 32)
            table_v[prev, 0, sl] = jnp.maximum(table_v[prev, 0, sl], accs[f])

    def _chunk(k, carry):
        prev_sc, prev, accs = carry
        s = pl.multiple_of(jnp.minimum(base + k * C, NR - C), 16)
        sc = lax.div(s, jnp.int32(IDS_W))
        soff = s - sc * IDS_W

        @pl.when(sc != prev_sc)
        def _():
            ssc = pl.multiple_of(sc * IDS_W, 128)
            pltpu.sync_copy(ids_hbm.at[:, pl.ds(ssc, IDS_W)], ids_v)

        pltpu.sync_copy(h2_hbm.at[:, pl.ds(s, C), :], rows_v)

        def _pair(jj, gcarry):
            prev, accs = gcarry
            for side in range(2):
                idvec = ids_v[2 * jj + side, pl.ds(soff, 16)]
                for r in range(C):
                    rel = idvec[r] - c_lo
                    rel1 = jnp.where(
                        jnp.logical_and(rel >= 0, rel < CPT), rel, CPT)
                    changed = rel1 != prev

                    @pl.when(changed)
                    def _(prev=prev, accs=accs):
                        _flush(prev, accs)

                    accs = tuple(
                        jnp.maximum(
                            jnp.where(changed, zero32, accs[f]),
                            rows_v[jj, r, pl.ds(64 * side + 32 * f, 32)])
                        for f in range(HID // 32))
                    prev = rel1
            return prev, accs

        prev, accs = lax.fori_loop(0, 16, _pair, (prev, accs))
        return sc, prev, accs

    _, prev, accs = lax.fori_loop(
        0, nk, _chunk,
        (jnp.int32(-1), jnp.int32(CPT), (zero32,) * (HID // 32)))
    _flush(prev, accs)
    pltpu.sync_copy(table_v.at[pl.ds(0, CPT), :, :],
                    out_hbm.at[pl.ds(c_lo, CPT), :, :])


def _run_segmax(h2, ids, bounds):
    mesh = plsc.VectorSubcoreMesh(core_axis_name="c", subcore_axis_name="s")
    f = functools.partial(
        pl.kernel,
        mesh=mesh,
        out_type=jax.ShapeDtypeStruct((NC_PAD, 2, HID), jnp.bfloat16),
        scratch_types=[
            pltpu.VMEM((48,), jnp.int32),
            pltpu.VMEM((P, IDS_W), jnp.int32),
            pltpu.VMEM((16, C, 128), jnp.bfloat16),
            pltpu.VMEM((TROWS, 2, HID), jnp.bfloat16),
        ],
    )(_segmax_body)
    return f(h2, ids, bounds)


def _head_body(xc_ref, w3_ref, b3_ref, cb_ref, o_ref):
    xc3 = jnp.dot(xc_ref[...].astype(jnp.float32), w3_ref[...],
                  preferred_element_type=jnp.float32)
    xc3 = xc3 + b3_ref[0:1, :]
    cb = cb_ref[0:1, :]
    iot = lax.broadcasted_iota(jnp.int32, (N_FOV, NC_PAD), 0)
    onehot = (iot == cb).astype(jnp.float32)
    sums = jnp.dot(onehot, xc3, preferred_element_type=jnp.float32)
    counts = jnp.sum(onehot, axis=1, keepdims=True)
    xfov = sums / jnp.maximum(counts, 1.0)
    logits = xfov[:, 0:N_CLASSES]
    m = jnp.max(logits, axis=1, keepdims=True)
    ls = (logits - m) - jnp.log(jnp.sum(jnp.exp(logits - m), axis=1, keepdims=True))
    o_ref[...] = ls


def _run_head(xc, w3p, b3r, cb2):
    return pl.pallas_call(
        _head_body,
        in_specs=[
            pl.BlockSpec((NC_PAD, 2 * HID), lambda: (0, 0)),
            pl.BlockSpec((2 * HID, 8), lambda: (0, 0)),
            pl.BlockSpec((8, 8), lambda: (0, 0)),
            pl.BlockSpec((8, NC_PAD), lambda: (0, 0)),
        ],
        out_specs=pl.BlockSpec((N_FOV, N_CLASSES), lambda: (0, 0)),
        out_shape=jax.ShapeDtypeStruct((N_FOV, N_CLASSES), jnp.float32),
    )(xc, w3p, b3r, cb2)


def kernel(x_locs, pos_locs, cluster_id, cluster_batch, W1, b1, W2, b2, W3, b3):
    targets = jnp.minimum(jnp.arange(NW + 1, dtype=jnp.int32) * CPT, N_CLUSTERS)
    bounds = jnp.searchsorted(cluster_id, targets,
                              method="compare_all").astype(jnp.int32)
    bounds = jnp.pad(bounds, (0, 48 - (NW + 1)))

    xr = x_locs.reshape(NR, 4 * P).astype(jnp.bfloat16)
    pr = pos_locs.reshape(NR, 3 * P).astype(jnp.bfloat16)
    ids = cluster_id.reshape(NR, P).T
    ids = jnp.pad(ids, ((0, 0), (0, NR_PAD - NR)), constant_values=1 << 20)

    eye32 = jnp.eye(P, dtype=jnp.float32)
    w1x = jnp.kron(eye32, W1[0:4]).astype(jnp.bfloat16)
    w1p = jnp.kron(eye32, W1[4:7]).astype(jnp.bfloat16)
    b1t = jnp.broadcast_to(jnp.tile(b1, P)[None, :], (8, HID * P))
    eye4 = jnp.eye(4, dtype=jnp.float32)
    w2bd = jnp.kron(eye4, W2).astype(jnp.bfloat16)
    b2t = jnp.broadcast_to(jnp.tile(b2, 4)[None, :], (8, 256))

    w3p = jnp.pad(W3, ((0, 0), (0, 8 - N_CLASSES)))
    w3w = jnp.concatenate([w3p, jnp.zeros((HID, 8), jnp.float32)])
    b3r = jnp.broadcast_to(jnp.pad(b3, (0, 8 - N_CLASSES))[None, :], (8, 8))
    cb_pad = jnp.concatenate(
        [cluster_batch, jnp.full((NC_PAD - N_CLUSTERS,), -1, jnp.int32)])
    cb2 = jnp.broadcast_to(cb_pad[None, :], (8, NC_PAD))

    h2 = _run_mlp(xr, pr, w1x, w1p, b1t, w2bd, b2t)
    xc2 = _run_segmax(h2, ids, bounds)
    xcw = xc2.reshape(NC_PAD, 2 * HID)
    return _run_head(xcw, w3w, b3r, cb2)


# speedup vs baseline: 2.2606x; 2.2606x over previous
import functools

import jax
import jax.numpy as jnp
from jax import lax
from jax.experimental import pallas as pl
from jax.experimental.pallas import tpu as pltpu
from jax.experimental.pallas import tpu_sc as plsc

N_LOCS = 640000
N_CLUSTERS = 10000
N_FOV = 16
HID = 64
N_CLASSES = 4

R1 = 5120
CHUNK = 512
NW = 32
CPT = 320
NC_PAD = NW * CPT
TROWS = CPT + 16


def _mlp_body(xt_ref, w1_ref, b1_ref, w2_ref, b2_ref, o_ref):
    h = lax.dot_general(xt_ref[...], w1_ref[...],
                        (((0,), (0,)), ((), ())),
                        preferred_element_type=jnp.float32)
    h = jnp.maximum(h + b1_ref[0:1, :], 0.0).astype(jnp.bfloat16)
    h = jnp.dot(h, w2_ref[...], preferred_element_type=jnp.float32)
    o_ref[...] = jnp.maximum(h + b2_ref[0:1, :], 0.0)


def _run_mlp(xt, w1b, b1r, w2b, b2r):
    grid = (N_LOCS // R1,)
    return pl.pallas_call(
        _mlp_body,
        grid=grid,
        in_specs=[
            pl.BlockSpec((7, R1), lambda i: (0, i)),
            pl.BlockSpec((7, HID), lambda i: (0, 0)),
            pl.BlockSpec((8, HID), lambda i: (0, 0)),
            pl.BlockSpec((HID, HID), lambda i: (0, 0)),
            pl.BlockSpec((8, HID), lambda i: (0, 0)),
        ],
        out_specs=pl.BlockSpec((R1, HID), lambda i: (i, 0)),
        out_shape=jax.ShapeDtypeStruct((N_LOCS, HID), jnp.float32),
    )(xt, w1b, b1r, w2b, b2r)


def _segmax_body(h2_hbm, ids_hbm, bounds_hbm, out_hbm, bounds_v, ids_v, rows_v, table_v):
    w = lax.axis_index("s") * 2 + lax.axis_index("c")
    pltpu.sync_copy(bounds_hbm, bounds_v)
    bv = bounds_v[pl.ds(w, 16)]
    row_lo = bv[0]
    row_hi = bv[1]
    base = lax.bitwise_and(row_lo, jnp.int32(-16))
    c_lo = w * CPT

    def _zero(i, carry):
        for f in range(HID // 16):
            table_v[i, pl.ds(16 * f, 16)] = jnp.zeros((16,), jnp.float32)
        return carry

    lax.fori_loop(0, TROWS, _zero, 0)

    nk = (row_hi - base + (CHUNK - 1)) // CHUNK
    zero16 = jnp.zeros((16,), jnp.float32)

    def _flush(prev, accs):
        for f in range(HID // 16):
            sl = pl.ds(16 * f, 16)
            table_v[prev, sl] = jnp.maximum(table_v[prev, sl], accs[f])

    def _chunk(k, carry):
        s = pl.multiple_of(jnp.minimum(base + k * CHUNK, N_LOCS - CHUNK), 16)
        pltpu.sync_copy(ids_hbm.at[pl.ds(s, CHUNK)], ids_v)
        pltpu.sync_copy(h2_hbm.at[pl.ds(s, CHUNK), :], rows_v)

        def _grp(g, gcarry):
            prev, accs = gcarry
            idvec = ids_v[pl.ds(g * 16, 16)]
            for j in range(16):
                rel = idvec[j] - c_lo
                rel1 = jnp.where(
                    jnp.logical_and(rel >= 0, rel < CPT), rel, CPT)
                changed = rel1 != prev

                @pl.when(changed)
                def _(prev=prev, accs=accs):
                    _flush(prev, accs)

                r = g * 16 + j
                accs = tuple(
                    jnp.maximum(jnp.where(changed, zero16, accs[f]),
                                rows_v[r, pl.ds(16 * f, 16)])
                    for f in range(HID // 16))
                prev = rel1
            return prev, accs

        return lax.fori_loop(0, CHUNK // 16, _grp, carry)

    prev, accs = lax.fori_loop(0, nk, _chunk,
                               (jnp.int32(CPT), (zero16,) * (HID // 16)))
    _flush(prev, accs)
    pltpu.sync_copy(table_v.at[pl.ds(0, CPT), :], out_hbm.at[pl.ds(c_lo, CPT), :])


def _run_segmax(h2, ids, bounds):
    mesh = plsc.VectorSubcoreMesh(core_axis_name="c", subcore_axis_name="s")
    f = functools.partial(
        pl.kernel,
        mesh=mesh,
        out_type=jax.ShapeDtypeStruct((NC_PAD, HID), jnp.float32),
        scratch_types=[
            pltpu.VMEM((48,), jnp.int32),
            pltpu.VMEM((CHUNK,), jnp.int32),
            pltpu.VMEM((CHUNK, HID), jnp.float32),
            pltpu.VMEM((TROWS, HID), jnp.float32),
        ],
    )(_segmax_body)
    return f(h2, ids, bounds)


def _head_body(xc_ref, w3_ref, b3_ref, cb_ref, o_ref):
    xc3 = jnp.dot(xc_ref[...], w3_ref[...], preferred_element_type=jnp.float32)
    xc3 = xc3 + b3_ref[0:1, :]
    cb = cb_ref[0:1, :]
    iot = lax.broadcasted_iota(jnp.int32, (N_FOV, NC_PAD), 0)
    onehot = (iot == cb).astype(jnp.float32)
    sums = jnp.dot(onehot, xc3, preferred_element_type=jnp.float32)
    counts = jnp.sum(onehot, axis=1, keepdims=True)
    xfov = sums / jnp.maximum(counts, 1.0)
    logits = xfov[:, 0:N_CLASSES]
    m = jnp.max(logits, axis=1, keepdims=True)
    ls = (logits - m) - jnp.log(jnp.sum(jnp.exp(logits - m), axis=1, keepdims=True))
    o_ref[...] = ls


def _run_head(xc, w3p, b3r, cb2):
    return pl.pallas_call(
        _head_body,
        in_specs=[
            pl.BlockSpec((NC_PAD, HID), lambda: (0, 0)),
            pl.BlockSpec((HID, 8), lambda: (0, 0)),
            pl.BlockSpec((8, 8), lambda: (0, 0)),
            pl.BlockSpec((8, NC_PAD), lambda: (0, 0)),
        ],
        out_specs=pl.BlockSpec((N_FOV, N_CLASSES), lambda: (0, 0)),
        out_shape=jax.ShapeDtypeStruct((N_FOV, N_CLASSES), jnp.float32),
    )(xc, w3p, b3r, cb2)


def kernel(x_locs, pos_locs, cluster_id, cluster_batch, W1, b1, W2, b2, W3, b3):
    targets = jnp.minimum(jnp.arange(NW + 1, dtype=jnp.int32) * CPT, N_CLUSTERS)
    bounds = jnp.searchsorted(cluster_id, targets,
                              method="compare_all").astype(jnp.int32)
    bounds = jnp.pad(bounds, (0, 48 - (NW + 1)))

    xt = jnp.concatenate([x_locs.T, pos_locs.T], axis=0).astype(jnp.bfloat16)

    w1b = W1.astype(jnp.bfloat16)
    b1r = jnp.broadcast_to(b1[None, :], (8, HID))
    w2b = W2.astype(jnp.bfloat16)
    b2r = jnp.broadcast_to(b2[None, :], (8, HID))
    w3p = jnp.pad(W3, ((0, 0), (0, 8 - N_CLASSES)))
    b3r = jnp.broadcast_to(jnp.pad(b3, (0, 8 - N_CLASSES))[None, :], (8, 8))
    cb_pad = jnp.concatenate(
        [cluster_batch, jnp.full((NC_PAD - N_CLUSTERS,), -1, jnp.int32)])
    cb2 = jnp.broadcast_to(cb_pad[None, :], (8, NC_PAD))

    h2 = _run_mlp(xt, w1b, b1r, w2b, b2r)
    xc = _run_segmax(h2, cluster_id, bounds)
    return _run_head(xc, w3p, b3r, cb2)
